# baseline (speedup 1.0000x reference)
import functools

import jax
import jax.numpy as jnp
import numpy as np
from jax import lax
from jax.experimental import pallas as pl
from jax.experimental.pallas import tpu as pltpu
from jax.experimental.pallas import tpu_sc as plsc

N = 10000
DEG = 32
D = 128

NC = 2
NS = 16
NW = NC * NS
CH = 128

BN = 2000

SPLITS = (8, 8, 8, 8)


NBUF = 2


def _sc_gather_piece(table, idx3, nchunk):
    ch_base = nchunk // NW
    ch_extra = nchunk - ch_base * NW
    ch_max = idx3.shape[1]
    mesh = plsc.VectorSubcoreMesh(core_axis_name="c", subcore_axis_name="s")

    @functools.partial(
        pl.kernel,
        mesh=mesh,
        out_type=jax.ShapeDtypeStruct((nchunk * CH, D), jnp.float32),
        scratch_types=[
            pltpu.VMEM((ch_max, CH), jnp.int32),
            pltpu.VMEM((NBUF, CH, D), jnp.float32),
            pltpu.SemaphoreType.DMA,
            pltpu.SemaphoreType.DMA,
        ],
    )
    def gather_kernel(table_hbm, idx_hbm, out_hbm, idx_v, buf, gsem, ssem):
        wid = lax.axis_index("s") * NC + lax.axis_index("c")
        nch = jnp.where(wid < ch_extra, ch_base + 1, ch_base)
        base_ch = wid * ch_base + jnp.minimum(wid, ch_extra)
        pltpu.sync_copy(idx_hbm.at[wid], idx_v)
        pltpu.async_copy(table_hbm.at[idx_v.at[0]], buf.at[0], gsem)

        def chunk_body(k, _):
            slot = lax.rem(k, 2)
            pltpu.make_async_copy(
                table_hbm.at[idx_v.at[k]], buf.at[slot], gsem
            ).wait()

            @pl.when(k + 1 < nch)
            def _():
                @pl.when(k >= 1)
                def _():
                    pltpu.make_async_copy(
                        buf.at[1 - slot], out_hbm.at[pl.ds(0, CH)], ssem
                    ).wait()

                pltpu.async_copy(
                    table_hbm.at[idx_v.at[k + 1]], buf.at[1 - slot], gsem
                )

            pltpu.async_copy(
                buf.at[slot], out_hbm.at[pl.ds((base_ch + k) * CH, CH)], ssem
            )
            return 0

        lax.fori_loop(0, nch, chunk_body, 0)
        pltpu.make_async_copy(buf.at[0], out_hbm.at[pl.ds(0, CH)], ssem).wait()
        pltpu.make_async_copy(buf.at[0], out_hbm.at[pl.ds(0, CH)], ssem).wait()

    return gather_kernel(table, idx3)


def _lstm_chunk(xg3, hprev, cprev, hin, wih_t, whh_t, bias2, wout, bout2,
                tg, first, last, relu):
    nb = N // BN

    def body(*refs):
        it = iter(refs)
        xg_ref = next(it)
        if not first:
            hp_ref = next(it)
            cp_ref = next(it)
        wih_ref = next(it)
        whh_ref = next(it)
        b_ref = next(it)
        if last:
            hin_ref = next(it)
            wout_ref = next(it)
            bo_ref = next(it)
            out_ref = next(it)
        else:
            ho_ref = next(it)
            co_ref = next(it)

        def sigm(v):
            return 0.5 + 0.5 * jnp.tanh(v)

        if first:
            h = jnp.zeros((BN, D), jnp.float32)
            c = jnp.zeros((BN, D), jnp.float32)
        else:
            h = hp_ref[...]
            c = cp_ref[...]
        for t in range(tg):
            xw = jnp.dot(xg_ref[t].astype(jnp.bfloat16), wih_ref[...],
                         preferred_element_type=jnp.float32)
            gates = xw + jnp.dot(h.astype(jnp.bfloat16), whh_ref[...],
                                 preferred_element_type=jnp.float32) + b_ref[...]
            gi = sigm(gates[:, 0:D])
            gf = sigm(gates[:, D:2 * D])
            gg = jnp.tanh(gates[:, 2 * D:3 * D])
            go = sigm(gates[:, 3 * D:4 * D])
            c = gf * c + gi * gg
            h = go * jnp.tanh(c)

        if last:
            cat2 = jnp.concatenate([hin_ref[...], h], axis=1)
            o2 = jnp.dot(cat2, wout_ref[...],
                         preferred_element_type=jnp.float32) + bo_ref[...]
            out_ref[...] = jnp.maximum(o2, 0.0) if relu else o2
        else:
            ho_ref[...] = h
            co_ref[...] = c

    blk_nd = pl.BlockSpec((BN, D), lambda b: (b, 0))
    in_specs = [pl.BlockSpec((tg, BN, D), lambda b: (0, b, 0))]
    inputs = [xg3]
    if not first:
        in_specs += [blk_nd, blk_nd]
        inputs += [hprev, cprev]
    in_specs += [
        pl.BlockSpec((D, 4 * D), lambda b: (0, 0)),
        pl.BlockSpec((D, 4 * D), lambda b: (0, 0)),
        pl.BlockSpec((1, 4 * D), lambda b: (0, 0)),
    ]
    inputs += [wih_t, whh_t, bias2]
    if last:
        in_specs += [
            blk_nd,
            pl.BlockSpec((2 * D, D), lambda b: (0, 0)),
            pl.BlockSpec((1, D), lambda b: (0, 0)),
        ]
        inputs += [hin, wout, bout2]
        out_specs = blk_nd
        out_shape = jax.ShapeDtypeStruct((N, D), jnp.float32)
    else:
        out_specs = (blk_nd, blk_nd)
        out_shape = (jax.ShapeDtypeStruct((N, D), jnp.float32),
                     jax.ShapeDtypeStruct((N, D), jnp.float32))

    return pl.pallas_call(
        body,
        grid=(nb,),
        in_specs=in_specs,
        out_specs=out_specs,
        out_shape=out_shape,
        compiler_params=pltpu.CompilerParams(
            dimension_semantics=("arbitrary",)),
    )(*inputs)


def _layer_weights(p):
    scale = jnp.concatenate(
        [jnp.full((2 * D,), 0.5), jnp.ones((D,)), jnp.full((D,), 0.5)]
    ).astype(jnp.float32)
    wih_t = (p["Wih"].T * scale).astype(jnp.bfloat16)
    whh_t = (p["Whh"].T * scale).astype(jnp.bfloat16)
    bias2 = ((p["bih"] + p["bhh"]) * scale).reshape(1, 4 * D)
    wout = jnp.concatenate([p["W_self"], p["W_neigh"]], axis=0)
    bout2 = p["b"].reshape(1, D)
    return wih_t, whh_t, bias2, wout, bout2


def _make_idx_pieces(src):
    flat = src.reshape(N, DEG).T.reshape(-1)
    pieces = []
    row0 = 0
    for tg in SPLITS:
        rows = tg * N
        assert rows % CH == 0
        nchunk = rows // CH
        chunks = flat[row0:row0 + rows].reshape(nchunk, CH)
        chunks = jnp.concatenate([chunks, jnp.zeros((1, CH), jnp.int32)])
        ch_base = nchunk // NW
        ch_extra = nchunk - ch_base * NW
        ch_max = (ch_base + 1 + 7) // 8 * 8
        row_map = np.full((NW, ch_max), nchunk, dtype=np.int32)
        for w in range(NW):
            nch_w = ch_base + (1 if w < ch_extra else 0)
            base_w = ch_base * w + min(w, ch_extra)
            row_map[w, :nch_w] = np.arange(base_w, base_w + nch_w)
        pieces.append((chunks[jnp.asarray(row_map)], nchunk))
        row0 += rows
    return pieces


def kernel(g_features, edge_index, params):
    src = edge_index[0].astype(jnp.int32)
    pieces = _make_idx_pieces(src)
    ngp = len(SPLITS)

    h = g_features
    for li, name in enumerate(("l1", "l2", "l3")):
        wih_t, whh_t, bias2, wout, bout2 = _layer_weights(params[name])
        hc = (None, None)
        out = None
        xgs = [_sc_gather_piece(h, idx3, nchunk) for idx3, nchunk in pieces]
        for g, tg in enumerate(SPLITS):
            res = _lstm_chunk(
                xgs[g].reshape(tg, N, D), hc[0], hc[1], h,
                wih_t, whh_t, bias2, wout, bout2,
                tg=tg, first=(g == 0), last=(g == ngp - 1), relu=(li < 2))
            if g == ngp - 1:
                out = res
            else:
                hc = res
        h = out
    return h


# speedup vs baseline: 1.1028x; 1.1028x over previous
import functools

import jax
import jax.numpy as jnp
import numpy as np
from jax import lax
from jax.experimental import pallas as pl
from jax.experimental.pallas import tpu as pltpu
from jax.experimental.pallas import tpu_sc as plsc

N = 10000
DEG = 32
D = 128

NC = 2
NS = 16
NW = NC * NS
CH = 128

BN = 1000

SPLITS = (8, 8, 8, 8)


NBUF = 2


def _sc_gather_piece(table, idx3, nchunk):
    ch_base = nchunk // NW
    ch_extra = nchunk - ch_base * NW
    ch_max = idx3.shape[1]
    mesh = plsc.VectorSubcoreMesh(core_axis_name="c", subcore_axis_name="s")

    @functools.partial(
        pl.kernel,
        mesh=mesh,
        out_type=jax.ShapeDtypeStruct((nchunk * CH, D), jnp.float32),
        scratch_types=[
            pltpu.VMEM((ch_max, CH), jnp.int32),
            pltpu.VMEM((NBUF, CH, D), jnp.float32),
            pltpu.SemaphoreType.DMA,
            pltpu.SemaphoreType.DMA,
        ],
    )
    def gather_kernel(table_hbm, idx_hbm, out_hbm, idx_v, buf, gsem, ssem):
        wid = lax.axis_index("s") * NC + lax.axis_index("c")
        nch = jnp.where(wid < ch_extra, ch_base + 1, ch_base)
        base_ch = wid * ch_base + jnp.minimum(wid, ch_extra)
        pltpu.sync_copy(idx_hbm.at[wid], idx_v)
        pltpu.async_copy(table_hbm.at[idx_v.at[0]], buf.at[0], gsem)

        def chunk_body(k, _):
            slot = lax.rem(k, 2)
            pltpu.make_async_copy(
                table_hbm.at[idx_v.at[k]], buf.at[slot], gsem
            ).wait()

            @pl.when(k + 1 < nch)
            def _():
                @pl.when(k >= 1)
                def _():
                    pltpu.make_async_copy(
                        buf.at[1 - slot], out_hbm.at[pl.ds(0, CH)], ssem
                    ).wait()

                pltpu.async_copy(
                    table_hbm.at[idx_v.at[k + 1]], buf.at[1 - slot], gsem
                )

            pltpu.async_copy(
                buf.at[slot], out_hbm.at[pl.ds((base_ch + k) * CH, CH)], ssem
            )
            return 0

        lax.fori_loop(0, nch, chunk_body, 0)
        pltpu.make_async_copy(buf.at[0], out_hbm.at[pl.ds(0, CH)], ssem).wait()
        pltpu.make_async_copy(buf.at[0], out_hbm.at[pl.ds(0, CH)], ssem).wait()

    return gather_kernel(table, idx3)


def _lstm_chunk(xg3, hprev, cprev, hin, wih_t, whh_t, bias2, wout, bout2,
                tg, first, last, relu):
    nb = N // BN

    def body(*refs):
        it = iter(refs)
        xg_ref = next(it)
        if not first:
            hp_ref = next(it)
            cp_ref = next(it)
        wih_ref = next(it)
        whh_ref = next(it)
        b_ref = next(it)
        if last:
            hin_ref = next(it)
            wout_ref = next(it)
            bo_ref = next(it)
            out_ref = next(it)
        else:
            ho_ref = next(it)
            co_ref = next(it)

        def sigm(v):
            return 0.5 + 0.5 * jnp.tanh(v)

        if first:
            h = jnp.zeros((BN, D), jnp.float32)
            c = jnp.zeros((BN, D), jnp.float32)
        else:
            h = hp_ref[...]
            c = cp_ref[...]
        for t in range(tg):
            xw = jnp.dot(xg_ref[t], wih_ref[...],
                         preferred_element_type=jnp.float32)
            gates = xw + jnp.dot(h, whh_ref[...],
                                 preferred_element_type=jnp.float32) + b_ref[...]
            gi = sigm(gates[:, 0:D])
            gf = sigm(gates[:, D:2 * D])
            gg = jnp.tanh(gates[:, 2 * D:3 * D])
            go = sigm(gates[:, 3 * D:4 * D])
            c = gf * c + gi * gg
            h = go * jnp.tanh(c)

        if last:
            cat2 = jnp.concatenate([hin_ref[...], h], axis=1)
            o2 = jnp.dot(cat2, wout_ref[...],
                         preferred_element_type=jnp.float32) + bo_ref[...]
            out_ref[...] = jnp.maximum(o2, 0.0) if relu else o2
        else:
            ho_ref[...] = h
            co_ref[...] = c

    blk_nd = pl.BlockSpec((BN, D), lambda b: (b, 0))
    in_specs = [pl.BlockSpec((tg, BN, D), lambda b: (0, b, 0))]
    inputs = [xg3]
    if not first:
        in_specs += [blk_nd, blk_nd]
        inputs += [hprev, cprev]
    in_specs += [
        pl.BlockSpec((D, 4 * D), lambda b: (0, 0)),
        pl.BlockSpec((D, 4 * D), lambda b: (0, 0)),
        pl.BlockSpec((1, 4 * D), lambda b: (0, 0)),
    ]
    inputs += [wih_t, whh_t, bias2]
    if last:
        in_specs += [
            blk_nd,
            pl.BlockSpec((2 * D, D), lambda b: (0, 0)),
            pl.BlockSpec((1, D), lambda b: (0, 0)),
        ]
        inputs += [hin, wout, bout2]
        out_specs = blk_nd
        out_shape = jax.ShapeDtypeStruct((N, D), jnp.float32)
    else:
        out_specs = (blk_nd, blk_nd)
        out_shape = (jax.ShapeDtypeStruct((N, D), jnp.float32),
                     jax.ShapeDtypeStruct((N, D), jnp.float32))

    return pl.pallas_call(
        body,
        grid=(nb,),
        in_specs=in_specs,
        out_specs=out_specs,
        out_shape=out_shape,
        compiler_params=pltpu.CompilerParams(
            dimension_semantics=("arbitrary",)),
    )(*inputs)


def _layer_weights(p):
    scale = jnp.concatenate(
        [jnp.full((2 * D,), 0.5), jnp.ones((D,)), jnp.full((D,), 0.5)]
    ).astype(jnp.float32)
    wih_t = p["Wih"].T * scale
    whh_t = p["Whh"].T * scale
    bias2 = ((p["bih"] + p["bhh"]) * scale).reshape(1, 4 * D)
    wout = jnp.concatenate([p["W_self"], p["W_neigh"]], axis=0)
    bout2 = p["b"].reshape(1, D)
    return wih_t, whh_t, bias2, wout, bout2


def _make_idx_pieces(src):
    flat = src.reshape(N, DEG).T.reshape(-1)
    pieces = []
    row0 = 0
    for tg in SPLITS:
        rows = tg * N
        assert rows % CH == 0
        nchunk = rows // CH
        chunks = flat[row0:row0 + rows].reshape(nchunk, CH)
        chunks = jnp.concatenate([chunks, jnp.zeros((1, CH), jnp.int32)])
        ch_base = nchunk // NW
        ch_extra = nchunk - ch_base * NW
        ch_max = (ch_base + 1 + 7) // 8 * 8
        row_map = np.full((NW, ch_max), nchunk, dtype=np.int32)
        for w in range(NW):
            nch_w = ch_base + (1 if w < ch_extra else 0)
            base_w = ch_base * w + min(w, ch_extra)
            row_map[w, :nch_w] = np.arange(base_w, base_w + nch_w)
        pieces.append((chunks[jnp.asarray(row_map)], nchunk))
        row0 += rows
    return pieces


def kernel(g_features, edge_index, params):
    src = edge_index[0].astype(jnp.int32)
    pieces = _make_idx_pieces(src)
    ngp = len(SPLITS)

    h = g_features
    for li, name in enumerate(("l1", "l2", "l3")):
        wih_t, whh_t, bias2, wout, bout2 = _layer_weights(params[name])
        hc = (None, None)
        out = None
        xgs = [_sc_gather_piece(h, idx3, nchunk) for idx3, nchunk in pieces]
        for g, tg in enumerate(SPLITS):
            res = _lstm_chunk(
                xgs[g].reshape(tg, N, D), hc[0], hc[1], h,
                wih_t, whh_t, bias2, wout, bout2,
                tg=tg, first=(g == 0), last=(g == ngp - 1), relu=(li < 2))
            if g == ngp - 1:
                out = res
            else:
                hc = res
        h = out
    return h


# speedup vs baseline: 1.1186x; 1.0144x over previous
import functools

import jax
import jax.numpy as jnp
import numpy as np
from jax import lax
from jax.experimental import pallas as pl
from jax.experimental.pallas import tpu as pltpu
from jax.experimental.pallas import tpu_sc as plsc

N = 10000
DEG = 32
D = 128

NC = 2
NS = 16
NW = NC * NS
CH = 128

BN = 2000

SPLITS = (8, 8, 8, 8)


NBUF = 2


def _sc_gather_piece(table, idx3, nchunk):
    ch_base = nchunk // NW
    ch_extra = nchunk - ch_base * NW
    ch_max = idx3.shape[1]
    mesh = plsc.VectorSubcoreMesh(core_axis_name="c", subcore_axis_name="s")

    @functools.partial(
        pl.kernel,
        mesh=mesh,
        out_type=jax.ShapeDtypeStruct((nchunk * CH, D), jnp.float32),
        scratch_types=[
            pltpu.VMEM((ch_max, CH), jnp.int32),
            pltpu.VMEM((NBUF, CH, D), jnp.float32),
            pltpu.SemaphoreType.DMA,
            pltpu.SemaphoreType.DMA,
        ],
    )
    def gather_kernel(table_hbm, idx_hbm, out_hbm, idx_v, buf, gsem, ssem):
        wid = lax.axis_index("s") * NC + lax.axis_index("c")
        nch = jnp.where(wid < ch_extra, ch_base + 1, ch_base)
        base_ch = wid * ch_base + jnp.minimum(wid, ch_extra)
        pltpu.sync_copy(idx_hbm.at[wid], idx_v)
        pltpu.async_copy(table_hbm.at[idx_v.at[0]], buf.at[0], gsem)

        def chunk_body(k, _):
            slot = lax.rem(k, 2)
            pltpu.make_async_copy(
                table_hbm.at[idx_v.at[k]], buf.at[slot], gsem
            ).wait()

            @pl.when(k + 1 < nch)
            def _():
                @pl.when(k >= 1)
                def _():
                    pltpu.make_async_copy(
                        buf.at[1 - slot], out_hbm.at[pl.ds(0, CH)], ssem
                    ).wait()

                pltpu.async_copy(
                    table_hbm.at[idx_v.at[k + 1]], buf.at[1 - slot], gsem
                )

            pltpu.async_copy(
                buf.at[slot], out_hbm.at[pl.ds((base_ch + k) * CH, CH)], ssem
            )
            return 0

        lax.fori_loop(0, nch, chunk_body, 0)
        pltpu.make_async_copy(buf.at[0], out_hbm.at[pl.ds(0, CH)], ssem).wait()
        pltpu.make_async_copy(buf.at[0], out_hbm.at[pl.ds(0, CH)], ssem).wait()

    return gather_kernel(table, idx3)


def _lstm_chunk(xg3, hprev, cprev, hin, wih_t, whh_t, bias2, wout, bout2,
                tg, first, last, relu):
    nb = N // BN

    def body(*refs):
        it = iter(refs)
        xg_ref = next(it)
        if not first:
            hp_ref = next(it)
            cp_ref = next(it)
        wih_ref = next(it)
        whh_ref = next(it)
        b_ref = next(it)
        if last:
            hin_ref = next(it)
            wout_ref = next(it)
            bo_ref = next(it)
            out_ref = next(it)
        else:
            ho_ref = next(it)
            co_ref = next(it)

        def sigm(v):
            return 0.5 + 0.5 * jnp.tanh(v)

        if first:
            h = jnp.zeros((BN, D), jnp.float32)
            c = jnp.zeros((BN, D), jnp.float32)
        else:
            h = hp_ref[...]
            c = cp_ref[...]
        for t in range(tg):
            xw = jnp.dot(xg_ref[t], wih_ref[...],
                         preferred_element_type=jnp.float32)
            gates = xw + jnp.dot(h, whh_ref[...],
                                 preferred_element_type=jnp.float32) + b_ref[...]
            gi = sigm(gates[:, 0:D])
            gf = sigm(gates[:, D:2 * D])
            gg = jnp.tanh(gates[:, 2 * D:3 * D])
            go = sigm(gates[:, 3 * D:4 * D])
            c = gf * c + gi * gg
            h = go * jnp.tanh(c)

        if last:
            cat2 = jnp.concatenate([hin_ref[...], h], axis=1)
            o2 = jnp.dot(cat2, wout_ref[...],
                         preferred_element_type=jnp.float32) + bo_ref[...]
            out_ref[...] = jnp.maximum(o2, 0.0) if relu else o2
        else:
            ho_ref[...] = h
            co_ref[...] = c

    blk_nd = pl.BlockSpec((BN, D), lambda b: (b, 0))
    in_specs = [pl.BlockSpec((tg, BN, D), lambda b: (0, b, 0))]
    inputs = [xg3]
    if not first:
        in_specs += [blk_nd, blk_nd]
        inputs += [hprev, cprev]
    in_specs += [
        pl.BlockSpec((D, 4 * D), lambda b: (0, 0)),
        pl.BlockSpec((D, 4 * D), lambda b: (0, 0)),
        pl.BlockSpec((1, 4 * D), lambda b: (0, 0)),
    ]
    inputs += [wih_t, whh_t, bias2]
    if last:
        in_specs += [
            blk_nd,
            pl.BlockSpec((2 * D, D), lambda b: (0, 0)),
            pl.BlockSpec((1, D), lambda b: (0, 0)),
        ]
        inputs += [hin, wout, bout2]
        out_specs = blk_nd
        out_shape = jax.ShapeDtypeStruct((N, D), jnp.float32)
    else:
        out_specs = (blk_nd, blk_nd)
        out_shape = (jax.ShapeDtypeStruct((N, D), jnp.float32),
                     jax.ShapeDtypeStruct((N, D), jnp.float32))

    return pl.pallas_call(
        body,
        grid=(nb,),
        in_specs=in_specs,
        out_specs=out_specs,
        out_shape=out_shape,
        compiler_params=pltpu.CompilerParams(
            dimension_semantics=("arbitrary",)),
    )(*inputs)


def _layer_weights(p):
    scale = jnp.concatenate(
        [jnp.full((2 * D,), 0.5), jnp.ones((D,)), jnp.full((D,), 0.5)]
    ).astype(jnp.float32)
    wih_t = p["Wih"].T * scale
    whh_t = p["Whh"].T * scale
    bias2 = ((p["bih"] + p["bhh"]) * scale).reshape(1, 4 * D)
    wout = jnp.concatenate([p["W_self"], p["W_neigh"]], axis=0)
    bout2 = p["b"].reshape(1, D)
    return wih_t, whh_t, bias2, wout, bout2


def _make_idx_pieces(src):
    flat = src.reshape(N, DEG).T.reshape(-1)
    pieces = []
    row0 = 0
    for tg in SPLITS:
        rows = tg * N
        assert rows % CH == 0
        nchunk = rows // CH
        chunks = flat[row0:row0 + rows].reshape(nchunk, CH)
        chunks = jnp.concatenate([chunks, jnp.zeros((1, CH), jnp.int32)])
        ch_base = nchunk // NW
        ch_extra = nchunk - ch_base * NW
        ch_max = (ch_base + 1 + 7) // 8 * 8
        row_map = np.full((NW, ch_max), nchunk, dtype=np.int32)
        for w in range(NW):
            nch_w = ch_base + (1 if w < ch_extra else 0)
            base_w = ch_base * w + min(w, ch_extra)
            row_map[w, :nch_w] = np.arange(base_w, base_w + nch_w)
        pieces.append((chunks[jnp.asarray(row_map)], nchunk))
        row0 += rows
    return pieces


def kernel(g_features, edge_index, params):
    src = edge_index[0].astype(jnp.int32)
    pieces = _make_idx_pieces(src)
    ngp = len(SPLITS)

    h = g_features
    for li, name in enumerate(("l1", "l2", "l3")):
        wih_t, whh_t, bias2, wout, bout2 = _layer_weights(params[name])
        hc = (None, None)
        out = None
        xgs = [_sc_gather_piece(h, idx3, nchunk) for idx3, nchunk in pieces]
        for g, tg in enumerate(SPLITS):
            res = _lstm_chunk(
                xgs[g].reshape(tg, N, D), hc[0], hc[1], h,
                wih_t, whh_t, bias2, wout, bout2,
                tg=tg, first=(g == 0), last=(g == ngp - 1), relu=(li < 2))
            if g == ngp - 1:
                out = res
            else:
                hc = res
        h = out
    return h
